# baseline (speedup 1.0000x reference)
import functools

import jax
import jax.numpy as jnp
from jax import lax
from jax.experimental import pallas as pl
from jax.experimental.pallas import tpu as pltpu
from jax.experimental.pallas import tpu_sc as plsc

_B, _N, _D = 16, 2048, 256
_ROWS = _B * _N
_NW = 32
_RPW = _ROWS // _NW
_CHUNK = 32
_NCH = _RPW // _CHUNK
_NSET = 4
_LANES = 16
_GRP = _D // _LANES
_REP = 2
_VROWS = 513

_BUF = [pltpu.VMEM((_CHUNK, _D), jnp.float32)] * (3 * _NSET)
_SEM = [pltpu.SemaphoreType.DMA] * (4 * _NSET + 1)


@functools.partial(
    pl.kernel,
    mesh=plsc.VectorSubcoreMesh(core_axis_name="c", subcore_axis_name="s"),
    out_type=jax.ShapeDtypeStruct(((_N + 1) * _B, _D), jnp.float32),
    scratch_types=[
        pltpu.VMEM((_RPW,), jnp.int32),
        pltpu.VMEM((_RPW,), jnp.int32),
        pltpu.VMEM((_NCH, _CHUNK), jnp.int32),
        pltpu.VMEM((_LANES, _D), jnp.float32),
    ] + _BUF + _SEM,
)
def _sc_node_feature(feat_hbm, idxin_hbm, idxout_hbm, inw_hbm, outw_hbm,
                     gt_hbm, out_hbm,
                     idxin_v, idxout_v, orow_v, gt_v, *bufs_and_sems):
    bufs = bufs_and_sems[:3 * _NSET]
    sems = bufs_and_sems[3 * _NSET:]
    sgt = sems[4 * _NSET]
    sets = tuple(
        (bufs[3 * i], bufs[3 * i + 1], bufs[3 * i + 2],
         sems[4 * i], sems[4 * i + 1], sems[4 * i + 2], sems[4 * i + 3])
        for i in range(_NSET))

    c = lax.axis_index("c")
    s = lax.axis_index("s")
    wid = s * 2 + c
    base = wid * _RPW
    batch = wid // 2
    half = wid % 2
    n0 = half * _RPW
    orow0 = 1 + n0

    pltpu.sync_copy(idxin_hbm.at[pl.ds(base, _RPW)], idxin_v)
    pltpu.sync_copy(idxout_hbm.at[pl.ds(base, _RPW)], idxout_v)

    rep_off = (wid % _REP) * _VROWS

    def rep_body(i, carry):
        sl = pl.ds(i * _LANES, _LANES)
        idxin_v[sl] = idxin_v[sl] + rep_off
        idxout_v[sl] = idxout_v[sl] + rep_off
        return carry
    lax.fori_loop(0, _RPW // _LANES, rep_body, 0)

    lane = lax.iota(jnp.int32, _LANES)

    def orow_body(k, carry):
        for jj in range(_CHUNK // _LANES):
            orow_v[k, pl.ds(jj * _LANES, _LANES)] = (
                (orow0 + k * _CHUNK + jj * _LANES + lane) * _B + batch)
        return carry
    lax.fori_loop(0, _NCH, orow_body, 0)

    @pl.when(wid == 0)
    def _():
        pltpu.sync_copy(gt_hbm, gt_v.at[pl.ds(0, 1)])
        for j in range(_GRP):
            sl = pl.ds(j * _LANES, _LANES)
            row = gt_v[0, sl]
            for r in range(1, _LANES):
                gt_v[r, sl] = row
        pltpu.async_copy(gt_v, out_hbm.at[lane], sgt).wait()

    def start_in(ci, st):
        fb, ib, ob = st[0], st[1], st[2]
        sf, si, so = st[3], st[4], st[5]
        pltpu.async_copy(
            feat_hbm.at[batch].at[pl.ds(n0 + ci * _CHUNK, _CHUNK)], fb, sf)
        pltpu.async_copy(inw_hbm.at[idxin_v.at[pl.ds(ci * _CHUNK, _CHUNK)]],
                         ib, si)
        pltpu.async_copy(outw_hbm.at[idxout_v.at[pl.ds(ci * _CHUNK, _CHUNK)]],
                         ob, so)

    def wait_in(ci, st):
        fb, ib, ob = st[0], st[1], st[2]
        sf, si, so = st[3], st[4], st[5]
        pltpu.make_async_copy(
            feat_hbm.at[batch].at[pl.ds(n0 + ci * _CHUNK, _CHUNK)],
            fb, sf).wait()
        pltpu.make_async_copy(
            inw_hbm.at[idxin_v.at[pl.ds(ci * _CHUNK, _CHUNK)]],
            ib, si).wait()
        pltpu.make_async_copy(
            outw_hbm.at[idxout_v.at[pl.ds(ci * _CHUNK, _CHUNK)]],
            ob, so).wait()

    def out_ref(ci):
        return out_hbm.at[orow_v.at[ci]]

    def wait_store(ci, st):
        pltpu.make_async_copy(st[0], out_ref(ci), st[6]).wait()

    def compute(st):
        fb, ib, ob = st[0], st[1], st[2]

        def row_body(r, carry):
            for j in range(_GRP):
                sl = pl.ds(j * _LANES, _LANES)
                fb[r, sl] = fb[r, sl] + ib[r, sl] + ob[r, sl]
            return carry
        lax.fori_loop(0, _CHUNK, row_body, 0)

    for b in range(_NSET):
        start_in(b, sets[b])

    def group_body(p, carry):
        for b in range(_NSET):
            ci = p * _NSET + b
            st = sets[b]
            wait_in(ci, st)
            compute(st)
            pltpu.async_copy(st[0], out_ref(ci), st[6])
            nxt = ci + _NSET - 1
            st_n = sets[(b + _NSET - 1) % _NSET]
            if b == 0:
                @pl.when(p > 0)
                def _():
                    wait_store(ci - 1, st_n)
                    start_in(nxt, st_n)
            else:
                @pl.when(p < _NCH // _NSET - 1)
                def _():
                    wait_store(ci - 1, st_n)
                    start_in(nxt, st_n)
        return carry
    lax.fori_loop(0, _NCH // _NSET, group_body, 0)

    for b in range(_NSET):
        ci = _NCH - _NSET + b
        wait_store(ci, sets[b])


def kernel(features, in_degree, out_degree, in_w, out_w, graph_token):
    idx_in = in_degree.astype(jnp.int32).reshape(_ROWS)
    idx_out = out_degree.astype(jnp.int32).reshape(_ROWS)
    in_w_rep = jnp.tile(in_w, (_REP, 1))
    out_w_rep = jnp.tile(out_w, (_REP, 1))
    out = _sc_node_feature(features, idx_in, idx_out, in_w_rep, out_w_rep,
                           graph_token)
    return out.reshape(_N + 1, _B, _D).transpose(1, 0, 2)


# speedup vs baseline: 1.0330x; 1.0330x over previous
import functools

import jax
import jax.numpy as jnp
from jax import lax
from jax.experimental import pallas as pl
from jax.experimental.pallas import tpu as pltpu
from jax.experimental.pallas import tpu_sc as plsc

_B, _N, _D = 16, 2048, 256
_ROWS = _B * _N
_NW = 32
_RPW = _ROWS // _NW
_CHUNK = 32
_NCH = _RPW // _CHUNK
_NSET = 4
_LANES = 16
_GRP = _D // _LANES
_REP = 4
_VROWS = 513

_BUF = [pltpu.VMEM((_CHUNK, _D), jnp.float32)] * (3 * _NSET)
_SEM = [pltpu.SemaphoreType.DMA] * (4 * _NSET + 1)


@functools.partial(
    pl.kernel,
    mesh=plsc.VectorSubcoreMesh(core_axis_name="c", subcore_axis_name="s"),
    out_type=jax.ShapeDtypeStruct(((_N + 1) * _B, _D), jnp.float32),
    scratch_types=[
        pltpu.VMEM((_RPW,), jnp.int32),
        pltpu.VMEM((_RPW,), jnp.int32),
        pltpu.VMEM((_NCH, _CHUNK), jnp.int32),
        pltpu.VMEM((_LANES, _D), jnp.float32),
    ] + _BUF + _SEM,
)
def _sc_node_feature(feat_hbm, idxin_hbm, idxout_hbm, inw_hbm, outw_hbm,
                     gt_hbm, out_hbm,
                     idxin_v, idxout_v, orow_v, gt_v, *bufs_and_sems):
    bufs = bufs_and_sems[:3 * _NSET]
    sems = bufs_and_sems[3 * _NSET:]
    sgt = sems[4 * _NSET]
    sets = tuple(
        (bufs[3 * i], bufs[3 * i + 1], bufs[3 * i + 2],
         sems[4 * i], sems[4 * i + 1], sems[4 * i + 2], sems[4 * i + 3])
        for i in range(_NSET))

    c = lax.axis_index("c")
    s = lax.axis_index("s")
    wid = s * 2 + c
    base = wid * _RPW
    batch = wid // 2
    half = wid % 2
    n0 = half * _RPW
    orow0 = 1 + n0

    pltpu.sync_copy(idxin_hbm.at[pl.ds(base, _RPW)], idxin_v)
    pltpu.sync_copy(idxout_hbm.at[pl.ds(base, _RPW)], idxout_v)

    rep_off = (wid % _REP) * _VROWS

    def rep_body(i, carry):
        sl = pl.ds(i * _LANES, _LANES)
        idxin_v[sl] = idxin_v[sl] + rep_off
        idxout_v[sl] = idxout_v[sl] + rep_off
        return carry
    lax.fori_loop(0, _RPW // _LANES, rep_body, 0)

    lane = lax.iota(jnp.int32, _LANES)

    def orow_body(k, carry):
        for jj in range(_CHUNK // _LANES):
            orow_v[k, pl.ds(jj * _LANES, _LANES)] = (
                (orow0 + k * _CHUNK + jj * _LANES + lane) * _B + batch)
        return carry
    lax.fori_loop(0, _NCH, orow_body, 0)

    @pl.when(wid == 0)
    def _():
        pltpu.sync_copy(gt_hbm, gt_v.at[pl.ds(0, 1)])
        for j in range(_GRP):
            sl = pl.ds(j * _LANES, _LANES)
            row = gt_v[0, sl]
            for r in range(1, _LANES):
                gt_v[r, sl] = row
        pltpu.async_copy(gt_v, out_hbm.at[lane], sgt).wait()

    def start_in(ci, st):
        fb, ib, ob = st[0], st[1], st[2]
        sf, si, so = st[3], st[4], st[5]
        pltpu.async_copy(
            feat_hbm.at[batch].at[pl.ds(n0 + ci * _CHUNK, _CHUNK)], fb, sf)
        pltpu.async_copy(inw_hbm.at[idxin_v.at[pl.ds(ci * _CHUNK, _CHUNK)]],
                         ib, si)
        pltpu.async_copy(outw_hbm.at[idxout_v.at[pl.ds(ci * _CHUNK, _CHUNK)]],
                         ob, so)

    def wait_in(ci, st):
        fb, ib, ob = st[0], st[1], st[2]
        sf, si, so = st[3], st[4], st[5]
        pltpu.make_async_copy(
            feat_hbm.at[batch].at[pl.ds(n0 + ci * _CHUNK, _CHUNK)],
            fb, sf).wait()
        pltpu.make_async_copy(
            inw_hbm.at[idxin_v.at[pl.ds(ci * _CHUNK, _CHUNK)]],
            ib, si).wait()
        pltpu.make_async_copy(
            outw_hbm.at[idxout_v.at[pl.ds(ci * _CHUNK, _CHUNK)]],
            ob, so).wait()

    def out_ref(ci):
        return out_hbm.at[orow_v.at[ci]]

    def wait_store(ci, st):
        pltpu.make_async_copy(st[0], out_ref(ci), st[6]).wait()

    def compute(st):
        fb, ib, ob = st[0], st[1], st[2]

        def row_body(r, carry):
            for j in range(_GRP):
                sl = pl.ds(j * _LANES, _LANES)
                fb[r, sl] = fb[r, sl] + ib[r, sl] + ob[r, sl]
            return carry
        lax.fori_loop(0, _CHUNK, row_body, 0)

    for b in range(_NSET):
        start_in(b, sets[b])

    def group_body(p, carry):
        for b in range(_NSET):
            ci = p * _NSET + b
            st = sets[b]
            wait_in(ci, st)
            compute(st)
            pltpu.async_copy(st[0], out_ref(ci), st[6])
            nxt = ci + _NSET - 1
            st_n = sets[(b + _NSET - 1) % _NSET]
            if b == 0:
                @pl.when(p > 0)
                def _():
                    wait_store(ci - 1, st_n)
                    start_in(nxt, st_n)
            else:
                @pl.when(p < _NCH // _NSET - 1)
                def _():
                    wait_store(ci - 1, st_n)
                    start_in(nxt, st_n)
        return carry
    lax.fori_loop(0, _NCH // _NSET, group_body, 0)

    for b in range(_NSET):
        ci = _NCH - _NSET + b
        wait_store(ci, sets[b])


def kernel(features, in_degree, out_degree, in_w, out_w, graph_token):
    idx_in = in_degree.astype(jnp.int32).reshape(_ROWS)
    idx_out = out_degree.astype(jnp.int32).reshape(_ROWS)
    in_w_rep = jnp.tile(in_w, (_REP, 1))
    out_w_rep = jnp.tile(out_w, (_REP, 1))
    out = _sc_node_feature(features, idx_in, idx_out, in_w_rep, out_w_rep,
                           graph_token)
    return out.reshape(_N + 1, _B, _D).transpose(1, 0, 2)
